# baseline (speedup 1.0000x reference)
import functools

import jax
import jax.numpy as jnp
from jax import lax
from jax.experimental import pallas as pl
from jax.experimental.pallas import tpu as pltpu
from jax.experimental.pallas import tpu_sc as plsc

NUM_CLASSES = 8192
FEATURE_LEN = 256
BATCH = 16384

NCORE = 2
NSUB = 16
HALF = FEATURE_LEN // NCORE
CHUNK = 128
SAMPLES_PER_TILE = BATCH // NSUB
ROUNDS = SAMPLES_PER_TILE // CHUNK
ROWS_PER_TILE = NUM_CLASSES // NSUB


def _sc_segment_sums(feat_cat, batch_label, zero_blk):
    mesh = plsc.VectorSubcoreMesh(core_axis_name="c", subcore_axis_name="s")

    @functools.partial(
        pl.kernel,
        out_type=jax.ShapeDtypeStruct((NCORE * NUM_CLASSES, HALF),
                                      jnp.float32),
        mesh=mesh,
        scratch_types=[
            pltpu.VMEM((CHUNK, HALF), jnp.float32),
            pltpu.VMEM((CHUNK,), jnp.int32),
            pltpu.VMEM_SHARED((NUM_CLASSES, HALF), jnp.float32),
        ],
    )
    def seg(fcat_hbm, lab_hbm, zblk_hbm, sums_hbm, feat_v, idx_v, acc):
        c = lax.axis_index("c")
        s = lax.axis_index("s")
        row0 = s * ROWS_PER_TILE

        pltpu.sync_copy(zblk_hbm, feat_v)
        for k in range(ROWS_PER_TILE // CHUNK):
            pltpu.sync_copy(feat_v, acc.at[pl.ds(row0 + k * CHUNK, CHUNK)])

        plsc.subcore_barrier()

        for r in range(ROUNDS):
            smp = s * SAMPLES_PER_TILE + r * CHUNK
            pltpu.sync_copy(lab_hbm.at[pl.ds(smp, CHUNK)], idx_v)
            pltpu.sync_copy(fcat_hbm.at[pl.ds(c * BATCH + smp, CHUNK)],
                            feat_v)
            pltpu.sync_copy(feat_v, acc.at[idx_v], add=True)

        plsc.subcore_barrier()

        pltpu.sync_copy(acc.at[pl.ds(row0, ROWS_PER_TILE)],
                        sums_hbm.at[pl.ds(c * NUM_CLASSES + row0,
                                          ROWS_PER_TILE)])

    return seg(feat_cat, batch_label, zero_blk)


_TC_GRID = 8
_FBLK = BATCH // _TC_GRID
_SBLK = NUM_CLASSES // _TC_GRID
_LROWS = BATCH // 128


def _tc_body(f_ref, s0_ref, s1_ref, lab_ref, out_ref, acc_ref):
    i = pl.program_id(0)

    @pl.when(i == 0)
    def _():
        acc_ref[0] = 0.0

    f = f_ref[...]
    s0 = s0_ref[...]
    s1 = s1_ref[...]
    q = (jnp.sum(s0 * s0, axis=1, keepdims=True)
         + jnp.sum(s1 * s1, axis=1, keepdims=True))


    base = i * _SBLK
    classes = base + lax.broadcasted_iota(jnp.int32, (_SBLK, 1), 0)

    def count_row(k, n_acc):
        lab_row = lab_ref[k, :]
        eq = (classes == lab_row[None, :]).astype(jnp.float32)
        return n_acc + jnp.sum(eq, axis=1, keepdims=True)

    n = lax.fori_loop(0, _LROWS, count_row,
                      jnp.zeros((_SBLK, 1), jnp.float32))

    part = jnp.sum(f * f) - jnp.sum(q / jnp.maximum(n, 1.0))
    acc_ref[0] = acc_ref[0] + part

    @pl.when(i == _TC_GRID - 1)
    def _():
        out_ref[...] = jnp.full((1, 1),
                                acc_ref[0] * (1.0 / (BATCH * FEATURE_LEN)),
                                jnp.float32)


def _tc_reduce(batch_feature, s0, s1, labels2d):
    return pl.pallas_call(
        _tc_body,
        grid=(_TC_GRID,),
        in_specs=[
            pl.BlockSpec((_FBLK, FEATURE_LEN), lambda i: (i, 0)),
            pl.BlockSpec((_SBLK, HALF), lambda i: (i, 0)),
            pl.BlockSpec((_SBLK, HALF), lambda i: (i, 0)),
            pl.BlockSpec((_LROWS, 128), lambda i: (0, 0)),
        ],
        out_specs=pl.BlockSpec((1, 1), lambda i: (0, 0)),
        out_shape=jax.ShapeDtypeStruct((1, 1), jnp.float32),
        scratch_shapes=[pltpu.SMEM((1,), jnp.float32)],
    )(batch_feature, s0, s1, labels2d)


@jax.jit
def kernel(batch_feature, batch_label):
    feat_cat = jnp.concatenate(
        [batch_feature[:, :HALF], batch_feature[:, HALF:]], axis=0)
    zero_blk = jnp.zeros((CHUNK, HALF), jnp.float32)
    sums = _sc_segment_sums(feat_cat, batch_label, zero_blk)
    s0 = sums[:NUM_CLASSES]
    s1 = sums[NUM_CLASSES:]
    labels2d = batch_label.reshape(_LROWS, 128)
    loss = _tc_reduce(batch_feature, s0, s1, labels2d)
    return loss[0, 0]


# speedup vs baseline: 3.2460x; 3.2460x over previous
import functools

import jax
import jax.numpy as jnp
from jax import lax
from jax.experimental import pallas as pl
from jax.experimental.pallas import tpu as pltpu
from jax.experimental.pallas import tpu_sc as plsc

NUM_CLASSES = 8192
FEATURE_LEN = 256
BATCH = 16384

NCORE = 2
NSUB = 16
HALF = FEATURE_LEN // NCORE
CHUNK = 128
SAMPLES_PER_TILE = BATCH // NSUB
ROUNDS = SAMPLES_PER_TILE // CHUNK
ROWS_PER_TILE = NUM_CLASSES // NSUB
CNT_SAMPLES_PER_TILE = BATCH // (NCORE * NSUB)
CNT_ROUNDS = CNT_SAMPLES_PER_TILE // CHUNK


def _sc_sums_and_counts(feat_cat, batch_label, zero_blk, one_blk):
    mesh = plsc.VectorSubcoreMesh(core_axis_name="c", subcore_axis_name="s")

    @functools.partial(
        pl.kernel,
        out_type=(
            jax.ShapeDtypeStruct((NCORE * NUM_CLASSES, HALF), jnp.float32),
            jax.ShapeDtypeStruct((NCORE * NUM_CLASSES, HALF), jnp.float32),
        ),
        mesh=mesh,
        scratch_types=[
            pltpu.VMEM((CHUNK, HALF), jnp.float32),
            pltpu.VMEM((CHUNK,), jnp.int32),
            pltpu.VMEM_SHARED((NUM_CLASSES, HALF), jnp.float32),
        ],
    )
    def seg(fcat_hbm, lab_hbm, zblk_hbm, oblk_hbm, sums_hbm, cnt_hbm,
            feat_v, idx_v, acc):
        c = lax.axis_index("c")
        s = lax.axis_index("s")
        row0 = s * ROWS_PER_TILE

        pltpu.sync_copy(zblk_hbm, feat_v)
        for k in range(ROWS_PER_TILE // CHUNK):
            pltpu.sync_copy(feat_v, acc.at[pl.ds(row0 + k * CHUNK, CHUNK)])

        plsc.subcore_barrier()

        for r in range(ROUNDS):
            smp = s * SAMPLES_PER_TILE + r * CHUNK
            pltpu.sync_copy(lab_hbm.at[pl.ds(smp, CHUNK)], idx_v)
            pltpu.sync_copy(fcat_hbm.at[pl.ds(c * BATCH + smp, CHUNK)],
                            feat_v)
            pltpu.sync_copy(feat_v, acc.at[idx_v], add=True)

        plsc.subcore_barrier()

        pltpu.sync_copy(acc.at[pl.ds(row0, ROWS_PER_TILE)],
                        sums_hbm.at[pl.ds(c * NUM_CLASSES + row0,
                                          ROWS_PER_TILE)])

        plsc.subcore_barrier()

        pltpu.sync_copy(zblk_hbm, feat_v)
        for k in range(ROWS_PER_TILE // CHUNK):
            pltpu.sync_copy(feat_v, acc.at[pl.ds(row0 + k * CHUNK, CHUNK)])

        plsc.subcore_barrier()

        pltpu.sync_copy(oblk_hbm, feat_v)
        for r in range(CNT_ROUNDS):
            smp = (c * NSUB + s) * CNT_SAMPLES_PER_TILE + r * CHUNK
            pltpu.sync_copy(lab_hbm.at[pl.ds(smp, CHUNK)], idx_v)
            pltpu.sync_copy(feat_v, acc.at[idx_v], add=True)

        plsc.subcore_barrier()

        pltpu.sync_copy(acc.at[pl.ds(row0, ROWS_PER_TILE)],
                        cnt_hbm.at[pl.ds(c * NUM_CLASSES + row0,
                                         ROWS_PER_TILE)])

    return seg(feat_cat, batch_label, zero_blk, one_blk)


_TC_GRID = 8
_FBLK = BATCH // _TC_GRID
_SBLK = NUM_CLASSES // _TC_GRID


def _tc_body(f_ref, s0_ref, s1_ref, c0_ref, c1_ref, out_ref, acc_ref):
    i = pl.program_id(0)

    @pl.when(i == 0)
    def _():
        acc_ref[0] = 0.0

    f = f_ref[...]
    s0 = s0_ref[...]
    s1 = s1_ref[...]
    q = (jnp.sum(s0 * s0, axis=1, keepdims=True)
         + jnp.sum(s1 * s1, axis=1, keepdims=True))
    n = c0_ref[:, 0:1] + c1_ref[:, 0:1]
    part = jnp.sum(f * f) - jnp.sum(q / jnp.maximum(n, 1.0))
    acc_ref[0] = acc_ref[0] + part

    @pl.when(i == _TC_GRID - 1)
    def _():
        out_ref[...] = jnp.full((1, 1),
                                acc_ref[0] * (1.0 / (BATCH * FEATURE_LEN)),
                                jnp.float32)


def _tc_reduce(batch_feature, sums, cnt):
    return pl.pallas_call(
        _tc_body,
        grid=(_TC_GRID,),
        in_specs=[
            pl.BlockSpec((_FBLK, FEATURE_LEN), lambda i: (i, 0)),
            pl.BlockSpec((_SBLK, HALF), lambda i: (i, 0)),
            pl.BlockSpec((_SBLK, HALF), lambda i: (i + _TC_GRID, 0)),
            pl.BlockSpec((_SBLK, HALF), lambda i: (i, 0)),
            pl.BlockSpec((_SBLK, HALF), lambda i: (i + _TC_GRID, 0)),
        ],
        out_specs=pl.BlockSpec((1, 1), lambda i: (0, 0)),
        out_shape=jax.ShapeDtypeStruct((1, 1), jnp.float32),
        scratch_shapes=[pltpu.SMEM((1,), jnp.float32)],
    )(batch_feature, sums, sums, cnt, cnt)


@jax.jit
def kernel(batch_feature, batch_label):
    feat_cat = jnp.concatenate(
        [batch_feature[:, :HALF], batch_feature[:, HALF:]], axis=0)
    zero_blk = jnp.zeros((CHUNK, HALF), jnp.float32)
    one_blk = jnp.ones((CHUNK, HALF), jnp.float32)
    sums, cnt = _sc_sums_and_counts(feat_cat, batch_label,
                                    zero_blk, one_blk)
    loss = _tc_reduce(batch_feature, sums, cnt)
    return loss[0, 0]


# speedup vs baseline: 3.9419x; 1.2144x over previous
import functools

import jax
import jax.numpy as jnp
from jax import lax
from jax.experimental import pallas as pl
from jax.experimental.pallas import tpu as pltpu
from jax.experimental.pallas import tpu_sc as plsc

NUM_CLASSES = 8192
FEATURE_LEN = 256
BATCH = 16384

NCORE = 2
NSUB = 16
HALF = FEATURE_LEN // NCORE
CHUNK = 128
SAMPLES_PER_TILE = BATCH // NSUB
ROUNDS = SAMPLES_PER_TILE // CHUNK
ROWS_PER_TILE = NUM_CLASSES // NSUB
CNT_SAMPLES_PER_TILE = BATCH // (NCORE * NSUB)
CNT_ROUNDS = CNT_SAMPLES_PER_TILE // CHUNK


def _sc_sums_and_counts(batch_feature, batch_label, zero_blk, one_blk):
    mesh = plsc.VectorSubcoreMesh(core_axis_name="c", subcore_axis_name="s")

    @functools.partial(
        pl.kernel,
        out_type=(
            jax.ShapeDtypeStruct((NCORE * NUM_CLASSES, HALF), jnp.float32),
            jax.ShapeDtypeStruct((NCORE * NUM_CLASSES, HALF), jnp.float32),
        ),
        mesh=mesh,
        scratch_types=[
            pltpu.VMEM((CHUNK, HALF), jnp.float32),
            pltpu.VMEM((CHUNK,), jnp.int32),
            pltpu.VMEM_SHARED((NUM_CLASSES, HALF), jnp.float32),
        ],
    )
    def seg(feat_hbm, lab_hbm, zblk_hbm, oblk_hbm, sums_hbm, cnt_hbm,
            feat_v, idx_v, acc):
        c = lax.axis_index("c")
        s = lax.axis_index("s")
        row0 = s * ROWS_PER_TILE

        pltpu.sync_copy(zblk_hbm, feat_v)
        for k in range(ROWS_PER_TILE // CHUNK):
            pltpu.sync_copy(feat_v, acc.at[pl.ds(row0 + k * CHUNK, CHUNK)])

        plsc.subcore_barrier()

        for r in range(ROUNDS):
            smp = s * SAMPLES_PER_TILE + r * CHUNK
            pltpu.sync_copy(lab_hbm.at[pl.ds(smp, CHUNK)], idx_v)
            pltpu.sync_copy(
                feat_hbm.at[pl.ds(smp, CHUNK), pl.ds(c * HALF, HALF)],
                feat_v)
            pltpu.sync_copy(feat_v, acc.at[idx_v], add=True)

        plsc.subcore_barrier()

        pltpu.sync_copy(acc.at[pl.ds(row0, ROWS_PER_TILE)],
                        sums_hbm.at[pl.ds(c * NUM_CLASSES + row0,
                                          ROWS_PER_TILE)])

        plsc.subcore_barrier()

        pltpu.sync_copy(zblk_hbm, feat_v)
        for k in range(ROWS_PER_TILE // CHUNK):
            pltpu.sync_copy(feat_v, acc.at[pl.ds(row0 + k * CHUNK, CHUNK)])

        plsc.subcore_barrier()

        pltpu.sync_copy(oblk_hbm, feat_v)
        for r in range(CNT_ROUNDS):
            smp = (c * NSUB + s) * CNT_SAMPLES_PER_TILE + r * CHUNK
            pltpu.sync_copy(lab_hbm.at[pl.ds(smp, CHUNK)], idx_v)
            pltpu.sync_copy(feat_v, acc.at[idx_v], add=True)

        plsc.subcore_barrier()

        pltpu.sync_copy(acc.at[pl.ds(row0, ROWS_PER_TILE)],
                        cnt_hbm.at[pl.ds(c * NUM_CLASSES + row0,
                                         ROWS_PER_TILE)])

    return seg(batch_feature, batch_label, zero_blk, one_blk)


_TC_GRID = 8
_FBLK = BATCH // _TC_GRID
_SBLK = NUM_CLASSES // _TC_GRID


def _tc_body(f_ref, s0_ref, s1_ref, c0_ref, c1_ref, out_ref, acc_ref):
    i = pl.program_id(0)

    @pl.when(i == 0)
    def _():
        acc_ref[0] = 0.0

    f = f_ref[...]
    s0 = s0_ref[...]
    s1 = s1_ref[...]
    q = (jnp.sum(s0 * s0, axis=1, keepdims=True)
         + jnp.sum(s1 * s1, axis=1, keepdims=True))
    n = c0_ref[:, 0:1] + c1_ref[:, 0:1]
    part = jnp.sum(f * f) - jnp.sum(q / jnp.maximum(n, 1.0))
    acc_ref[0] = acc_ref[0] + part

    @pl.when(i == _TC_GRID - 1)
    def _():
        out_ref[...] = jnp.full((1, 1),
                                acc_ref[0] * (1.0 / (BATCH * FEATURE_LEN)),
                                jnp.float32)


def _tc_reduce(batch_feature, sums, cnt):
    return pl.pallas_call(
        _tc_body,
        grid=(_TC_GRID,),
        in_specs=[
            pl.BlockSpec((_FBLK, FEATURE_LEN), lambda i: (i, 0)),
            pl.BlockSpec((_SBLK, HALF), lambda i: (i, 0)),
            pl.BlockSpec((_SBLK, HALF), lambda i: (i + _TC_GRID, 0)),
            pl.BlockSpec((_SBLK, HALF), lambda i: (i, 0)),
            pl.BlockSpec((_SBLK, HALF), lambda i: (i + _TC_GRID, 0)),
        ],
        out_specs=pl.BlockSpec((1, 1), lambda i: (0, 0)),
        out_shape=jax.ShapeDtypeStruct((1, 1), jnp.float32),
        scratch_shapes=[pltpu.SMEM((1,), jnp.float32)],
    )(batch_feature, sums, sums, cnt, cnt)


@jax.jit
def kernel(batch_feature, batch_label):
    zero_blk = jnp.zeros((CHUNK, HALF), jnp.float32)
    one_blk = jnp.ones((CHUNK, HALF), jnp.float32)
    sums, cnt = _sc_sums_and_counts(batch_feature, batch_label,
                                    zero_blk, one_blk)
    loss = _tc_reduce(batch_feature, sums, cnt)
    return loss[0, 0]


# speedup vs baseline: 4.0105x; 1.0174x over previous
import functools

import jax
import jax.numpy as jnp
from jax import lax
from jax.experimental import pallas as pl
from jax.experimental.pallas import tpu as pltpu
from jax.experimental.pallas import tpu_sc as plsc

NUM_CLASSES = 8192
FEATURE_LEN = 256
BATCH = 16384

NCORE = 2
NSUB = 16
HALF = FEATURE_LEN // NCORE
CHUNK = 256
GROUPS = CHUNK // 128
SAMPLES_PER_TILE = BATCH // NSUB
ROUNDS = SAMPLES_PER_TILE // CHUNK
ROWS_PER_TILE = NUM_CLASSES // NSUB
CNT_SAMPLES_PER_TILE = BATCH // (NCORE * NSUB)
CNT_ROUNDS = CNT_SAMPLES_PER_TILE // CHUNK


def _sc_sums_and_counts(batch_feature, batch_label, zero_blk, one_blk):
    mesh = plsc.VectorSubcoreMesh(core_axis_name="c", subcore_axis_name="s")

    @functools.partial(
        pl.kernel,
        out_type=(
            jax.ShapeDtypeStruct((NCORE * NUM_CLASSES, HALF), jnp.float32),
            jax.ShapeDtypeStruct((NCORE * NUM_CLASSES, HALF), jnp.float32),
        ),
        mesh=mesh,
        scratch_types=[
            pltpu.VMEM((CHUNK, HALF), jnp.float32),
            pltpu.VMEM((CHUNK,), jnp.int32),
            pltpu.VMEM_SHARED((NUM_CLASSES, HALF), jnp.float32),
        ],
    )
    def seg(feat_hbm, lab_hbm, zblk_hbm, oblk_hbm, sums_hbm, cnt_hbm,
            feat_v, idx_v, acc):
        c = lax.axis_index("c")
        s = lax.axis_index("s")
        row0 = s * ROWS_PER_TILE

        pltpu.sync_copy(zblk_hbm, feat_v)
        for k in range(ROWS_PER_TILE // CHUNK):
            pltpu.sync_copy(feat_v, acc.at[pl.ds(row0 + k * CHUNK, CHUNK)])

        plsc.subcore_barrier()

        for r in range(ROUNDS):
            smp = s * SAMPLES_PER_TILE + r * CHUNK
            pltpu.sync_copy(lab_hbm.at[pl.ds(smp, CHUNK)], idx_v)
            pltpu.sync_copy(
                feat_hbm.at[pl.ds(smp, CHUNK), pl.ds(c * HALF, HALF)],
                feat_v)
            for g in range(GROUPS):
                pltpu.sync_copy(feat_v.at[pl.ds(g * 128, 128)],
                                acc.at[idx_v.at[pl.ds(g * 128, 128)]],
                                add=True)

        plsc.subcore_barrier()

        pltpu.sync_copy(acc.at[pl.ds(row0, ROWS_PER_TILE)],
                        sums_hbm.at[pl.ds(c * NUM_CLASSES + row0,
                                          ROWS_PER_TILE)])

        plsc.subcore_barrier()

        pltpu.sync_copy(zblk_hbm, feat_v)
        for k in range(ROWS_PER_TILE // CHUNK):
            pltpu.sync_copy(feat_v, acc.at[pl.ds(row0 + k * CHUNK, CHUNK)])

        plsc.subcore_barrier()

        pltpu.sync_copy(oblk_hbm, feat_v)
        for r in range(CNT_ROUNDS):
            smp = (c * NSUB + s) * CNT_SAMPLES_PER_TILE + r * CHUNK
            pltpu.sync_copy(lab_hbm.at[pl.ds(smp, CHUNK)], idx_v)
            for g in range(GROUPS):
                pltpu.sync_copy(feat_v.at[pl.ds(g * 128, 128)],
                                acc.at[idx_v.at[pl.ds(g * 128, 128)]],
                                add=True)

        plsc.subcore_barrier()

        pltpu.sync_copy(acc.at[pl.ds(row0, ROWS_PER_TILE)],
                        cnt_hbm.at[pl.ds(c * NUM_CLASSES + row0,
                                         ROWS_PER_TILE)])

    return seg(batch_feature, batch_label, zero_blk, one_blk)


_TC_GRID = 8
_FBLK = BATCH // _TC_GRID
_SBLK = NUM_CLASSES // _TC_GRID


def _tc_body(f_ref, s0_ref, s1_ref, c0_ref, c1_ref, out_ref, acc_ref):
    i = pl.program_id(0)

    @pl.when(i == 0)
    def _():
        acc_ref[0] = 0.0

    f = f_ref[...]
    s0 = s0_ref[...]
    s1 = s1_ref[...]
    q = (jnp.sum(s0 * s0, axis=1, keepdims=True)
         + jnp.sum(s1 * s1, axis=1, keepdims=True))
    n = c0_ref[:, 0:1] + c1_ref[:, 0:1]
    part = jnp.sum(f * f) - jnp.sum(q / jnp.maximum(n, 1.0))
    acc_ref[0] = acc_ref[0] + part

    @pl.when(i == _TC_GRID - 1)
    def _():
        out_ref[...] = jnp.full((1, 1),
                                acc_ref[0] * (1.0 / (BATCH * FEATURE_LEN)),
                                jnp.float32)


def _tc_reduce(batch_feature, sums, cnt):
    return pl.pallas_call(
        _tc_body,
        grid=(_TC_GRID,),
        in_specs=[
            pl.BlockSpec((_FBLK, FEATURE_LEN), lambda i: (i, 0)),
            pl.BlockSpec((_SBLK, HALF), lambda i: (i, 0)),
            pl.BlockSpec((_SBLK, HALF), lambda i: (i + _TC_GRID, 0)),
            pl.BlockSpec((_SBLK, HALF), lambda i: (i, 0)),
            pl.BlockSpec((_SBLK, HALF), lambda i: (i + _TC_GRID, 0)),
        ],
        out_specs=pl.BlockSpec((1, 1), lambda i: (0, 0)),
        out_shape=jax.ShapeDtypeStruct((1, 1), jnp.float32),
        scratch_shapes=[pltpu.SMEM((1,), jnp.float32)],
    )(batch_feature, sums, sums, cnt, cnt)


@jax.jit
def kernel(batch_feature, batch_label):
    zero_blk = jnp.zeros((CHUNK, HALF), jnp.float32)
    one_blk = jnp.ones((CHUNK, HALF), jnp.float32)
    sums, cnt = _sc_sums_and_counts(batch_feature, batch_label,
                                    zero_blk, one_blk)
    loss = _tc_reduce(batch_feature, sums, cnt)
    return loss[0, 0]
